# baseline (speedup 1.0000x reference)
import functools

import jax
import jax.numpy as jnp
from jax.experimental import pallas as pl
from jax.experimental.pallas import tpu as pltpu

T = 2048
D = 1024
F = 4096
E = 8
K = 2

BLK = 256
P = K * T + E * BLK
NB = P // BLK
FB = 2048
NJ = F // FB

_SQRT_HALF = 0.7071067811865476


def _erf(z):
    a1, a2, a3, a4, a5 = (0.254829592, -0.284496736, 1.421413741,
                          -1.453152027, 1.061405429)
    s = jnp.sign(z)
    za = jnp.abs(z)
    t = 1.0 / (1.0 + 0.3275911 * za)
    poly = t * (a1 + t * (a2 + t * (a3 + t * (a4 + t * a5))))
    return s * (1.0 - poly * jnp.exp(-za * za))


def _gelu(h):
    return 0.5 * h * (1.0 + _erf(h * _SQRT_HALF))


def _ffn_body(be_ref, xs_ref, w1_ref, b1_ref, w2_ref, b2_ref, out_ref,
              w1b_ref, w2b_ref):
    j = pl.program_id(0)
    i = pl.program_id(1)
    changed = (i == 0) | (be_ref[i] != be_ref[jnp.maximum(i - 1, 0)])

    @pl.when(changed)
    def _recast():
        w1b_ref[...] = w1_ref[0].astype(jnp.bfloat16)
        w2b_ref[...] = w2_ref[0].astype(jnp.bfloat16)

    h = jnp.dot(xs_ref[...], w1b_ref[...], preferred_element_type=jnp.float32)
    h = _gelu(h + b1_ref[0, 0]).astype(jnp.bfloat16)
    y = jnp.dot(h, w2b_ref[...], preferred_element_type=jnp.float32)
    out_ref[0] = jnp.where(j == 0, y + b2_ref[0, 0], y)


@jax.jit
def _ffn(xs, block_e, W1, b1, W2, b2):
    grid_spec = pltpu.PrefetchScalarGridSpec(
        num_scalar_prefetch=1,
        grid=(NJ, NB),
        in_specs=[
            pl.BlockSpec((BLK, D), lambda j, i, be: (i, 0)),
            pl.BlockSpec((1, D, FB), lambda j, i, be: (be[i], 0, j)),
            pl.BlockSpec((1, 1, FB), lambda j, i, be: (be[i], 0, j)),
            pl.BlockSpec((1, FB, D), lambda j, i, be: (be[i], j, 0)),
            pl.BlockSpec((1, 1, D), lambda j, i, be: (be[i], 0, 0)),
        ],
        out_specs=pl.BlockSpec((1, BLK, D), lambda j, i, be: (j, i, 0)),
        scratch_shapes=[
            pltpu.VMEM((D, FB), jnp.bfloat16),
            pltpu.VMEM((FB, D), jnp.bfloat16),
        ],
    )
    return pl.pallas_call(
        _ffn_body,
        grid_spec=grid_spec,
        out_shape=jax.ShapeDtypeStruct((NJ, P, D), jnp.float32),
        compiler_params=pltpu.CompilerParams(
            dimension_semantics=("arbitrary", "arbitrary"),
        ),
    )(block_e, xs, W1, b1.reshape(E, 1, F), W2, b2.reshape(E, 1, D))


def kernel(x, gate_W, W1, b1, W2, b2):
    logits = x @ gate_W
    probs = jax.nn.softmax(logits, axis=-1)
    top_p, top_i = jax.lax.top_k(probs, K)
    top_w = top_p / jnp.sum(top_p, axis=-1, keepdims=True)

    ef = top_i.reshape(-1).astype(jnp.int32)
    order = jnp.argsort(ef)
    counts = jnp.sum(ef[None, :] == jnp.arange(E, dtype=jnp.int32)[:, None],
                     axis=1).astype(jnp.int32)
    blocks_e = (counts + BLK - 1) // BLK
    first_block = jnp.concatenate(
        [jnp.zeros((1,), jnp.int32), jnp.cumsum(blocks_e)[:-1]])
    pad_start = first_block * BLK
    seg_start = jnp.concatenate(
        [jnp.zeros((1,), jnp.int32), jnp.cumsum(counts)[:-1]])

    r = jnp.arange(K * T, dtype=jnp.int32)
    e_sorted = ef[order]
    padded_row = pad_start[e_sorted] + (r - seg_start[e_sorted])

    pos = jnp.zeros((K * T,), jnp.int32).at[order].set(padded_row)
    rowtok = jnp.zeros((P,), jnp.int32).at[padded_row].set(order // K)

    block_e = (jnp.sum(jnp.arange(NB, dtype=jnp.int32)[:, None]
                       >= first_block[None, :], axis=1) - 1).astype(jnp.int32)

    xs = jnp.take(x.astype(jnp.bfloat16), rowtok, axis=0)
    yp = _ffn(xs, block_e, W1, b1, W2, b2)

    pos2 = pos.reshape(T, K)
    y0 = jnp.take(yp[0], pos2[:, 0], axis=0) + jnp.take(yp[1], pos2[:, 0], axis=0)
    y1 = jnp.take(yp[0], pos2[:, 1], axis=0) + jnp.take(yp[1], pos2[:, 1], axis=0)
    return y0 * top_w[:, 0:1] + y1 * top_w[:, 1:2]


# speedup vs baseline: 1.0267x; 1.0267x over previous
import functools

import jax
import jax.numpy as jnp
from jax.experimental import pallas as pl
from jax.experimental.pallas import tpu as pltpu

T = 2048
D = 1024
F = 4096
E = 8
K = 2

BLK = 256
P = K * T + E * BLK
NB = P // BLK
FB = 2048
NJ = F // FB

_SQRT_HALF = 0.7071067811865476


def _erf(z):
    a1, a2, a3, a4, a5 = (0.254829592, -0.284496736, 1.421413741,
                          -1.453152027, 1.061405429)
    s = jnp.sign(z)
    za = jnp.abs(z)
    t = 1.0 / (1.0 + 0.3275911 * za)
    poly = t * (a1 + t * (a2 + t * (a3 + t * (a4 + t * a5))))
    return s * (1.0 - poly * jnp.exp(-za * za))


def _gelu(h):
    return 0.5 * h * (1.0 + _erf(h * _SQRT_HALF))


def _ffn_body(be_ref, xs_ref, w1_ref, b1_ref, w2_ref, b2_ref, out_ref):
    j = pl.program_id(0)
    w1 = w1_ref[0].astype(jnp.bfloat16)
    w2 = w2_ref[0].astype(jnp.bfloat16)
    h = jnp.dot(xs_ref[...], w1, preferred_element_type=jnp.float32)
    h = _gelu(h + b1_ref[0, 0]).astype(jnp.bfloat16)
    y = jnp.dot(h, w2, preferred_element_type=jnp.float32)
    out_ref[0] = jnp.where(j == 0, y + b2_ref[0, 0], y)


@jax.jit
def _ffn(xs, block_e, W1, b1, W2, b2):
    grid_spec = pltpu.PrefetchScalarGridSpec(
        num_scalar_prefetch=1,
        grid=(NJ, NB),
        in_specs=[
            pl.BlockSpec((BLK, D), lambda j, i, be: (i, 0)),
            pl.BlockSpec((1, D, FB), lambda j, i, be: (be[i], 0, j)),
            pl.BlockSpec((1, 1, FB), lambda j, i, be: (be[i], 0, j)),
            pl.BlockSpec((1, FB, D), lambda j, i, be: (be[i], j, 0)),
            pl.BlockSpec((1, 1, D), lambda j, i, be: (be[i], 0, 0)),
        ],
        out_specs=pl.BlockSpec((1, BLK, D), lambda j, i, be: (j, i, 0)),
    )
    return pl.pallas_call(
        _ffn_body,
        grid_spec=grid_spec,
        out_shape=jax.ShapeDtypeStruct((NJ, P, D), jnp.float32),
        compiler_params=pltpu.CompilerParams(
            dimension_semantics=("arbitrary", "arbitrary"),
        ),
    )(block_e, xs, W1, b1.reshape(E, 1, F), W2, b2.reshape(E, 1, D))


def kernel(x, gate_W, W1, b1, W2, b2):
    logits = x @ gate_W
    probs = jax.nn.softmax(logits, axis=-1)
    top_p, top_i = jax.lax.top_k(probs, K)
    top_w = top_p / jnp.sum(top_p, axis=-1, keepdims=True)

    ef = top_i.reshape(-1).astype(jnp.int32)
    order = jnp.argsort(ef)
    counts = jnp.sum(ef[None, :] == jnp.arange(E, dtype=jnp.int32)[:, None],
                     axis=1).astype(jnp.int32)
    blocks_e = (counts + BLK - 1) // BLK
    first_block = jnp.concatenate(
        [jnp.zeros((1,), jnp.int32), jnp.cumsum(blocks_e)[:-1]])
    pad_start = first_block * BLK
    seg_start = jnp.concatenate(
        [jnp.zeros((1,), jnp.int32), jnp.cumsum(counts)[:-1]])

    r = jnp.arange(K * T, dtype=jnp.int32)
    e_sorted = ef[order]
    padded_row = pad_start[e_sorted] + (r - seg_start[e_sorted])

    pos = jnp.zeros((K * T,), jnp.int32).at[order].set(padded_row)
    rowtok = jnp.zeros((P,), jnp.int32).at[padded_row].set(order // K)

    block_e = (jnp.sum(jnp.arange(NB, dtype=jnp.int32)[:, None]
                       >= first_block[None, :], axis=1) - 1).astype(jnp.int32)

    xs = jnp.take(x.astype(jnp.bfloat16), rowtok, axis=0)
    yp = _ffn(xs, block_e, W1, b1, W2, b2)

    pos2 = pos.reshape(T, K)
    y0 = jnp.take(yp[0], pos2[:, 0], axis=0) + jnp.take(yp[1], pos2[:, 0], axis=0)
    y1 = jnp.take(yp[0], pos2[:, 1], axis=0) + jnp.take(yp[1], pos2[:, 1], axis=0)
    return y0 * top_w[:, 0:1] + y1 * top_w[:, 1:2]


# speedup vs baseline: 1.2306x; 1.1986x over previous
import functools

import jax
import jax.numpy as jnp
from jax.experimental import pallas as pl
from jax.experimental.pallas import tpu as pltpu

T = 2048
D = 1024
F = 4096
E = 8
K = 2

BLK = 256
P = K * T + E * BLK
NB = P // BLK
FB = 2048
NJ = F // FB

_SQRT_HALF = 0.7071067811865476


def _gelu(h):
    c = 0.7978845608028654
    return 0.5 * h * (1.0 + jnp.tanh(c * (h + 0.044715 * h * h * h)))


def _ffn_body(be_ref, xs_ref, w1_ref, b1_ref, w2_ref, b2_ref, out_ref):
    j = pl.program_id(0)
    w1 = w1_ref[0].astype(jnp.bfloat16)
    w2 = w2_ref[0].astype(jnp.bfloat16)
    h = jnp.dot(xs_ref[...], w1, preferred_element_type=jnp.float32)
    h = _gelu(h + b1_ref[0, 0]).astype(jnp.bfloat16)
    y = jnp.dot(h, w2, preferred_element_type=jnp.float32)
    out_ref[0] = jnp.where(j == 0, y + b2_ref[0, 0], y)


@jax.jit
def _ffn(xs, block_e, W1, b1, W2, b2):
    grid_spec = pltpu.PrefetchScalarGridSpec(
        num_scalar_prefetch=1,
        grid=(NJ, NB),
        in_specs=[
            pl.BlockSpec((BLK, D), lambda j, i, be: (i, 0)),
            pl.BlockSpec((1, D, FB), lambda j, i, be: (be[i], 0, j)),
            pl.BlockSpec((1, 1, FB), lambda j, i, be: (be[i], 0, j)),
            pl.BlockSpec((1, FB, D), lambda j, i, be: (be[i], j, 0)),
            pl.BlockSpec((1, 1, D), lambda j, i, be: (be[i], 0, 0)),
        ],
        out_specs=pl.BlockSpec((1, BLK, D), lambda j, i, be: (j, i, 0)),
    )
    return pl.pallas_call(
        _ffn_body,
        grid_spec=grid_spec,
        out_shape=jax.ShapeDtypeStruct((NJ, P, D), jnp.float32),
        compiler_params=pltpu.CompilerParams(
            dimension_semantics=("arbitrary", "arbitrary"),
        ),
    )(block_e, xs, W1, b1.reshape(E, 1, F), W2, b2.reshape(E, 1, D))


def kernel(x, gate_W, W1, b1, W2, b2):
    logits = x @ gate_W
    probs = jax.nn.softmax(logits, axis=-1)
    top_p, top_i = jax.lax.top_k(probs, K)
    top_w = top_p / jnp.sum(top_p, axis=-1, keepdims=True)

    ef = top_i.reshape(-1).astype(jnp.int32)
    order = jnp.argsort(ef)
    counts = jnp.sum(ef[None, :] == jnp.arange(E, dtype=jnp.int32)[:, None],
                     axis=1).astype(jnp.int32)
    blocks_e = (counts + BLK - 1) // BLK
    first_block = jnp.concatenate(
        [jnp.zeros((1,), jnp.int32), jnp.cumsum(blocks_e)[:-1]])
    pad_start = first_block * BLK
    seg_start = jnp.concatenate(
        [jnp.zeros((1,), jnp.int32), jnp.cumsum(counts)[:-1]])

    r = jnp.arange(K * T, dtype=jnp.int32)
    e_sorted = ef[order]
    padded_row = pad_start[e_sorted] + (r - seg_start[e_sorted])

    pos = jnp.zeros((K * T,), jnp.int32).at[order].set(padded_row)
    rowtok = jnp.zeros((P,), jnp.int32).at[padded_row].set(order // K)

    block_e = (jnp.sum(jnp.arange(NB, dtype=jnp.int32)[:, None]
                       >= first_block[None, :], axis=1) - 1).astype(jnp.int32)

    xs = jnp.take(x.astype(jnp.bfloat16), rowtok, axis=0)
    yp = _ffn(xs, block_e, W1, b1, W2, b2)
    ys = jnp.sum(yp, axis=0)

    pos2 = pos.reshape(T, K)
    out = (jnp.take(ys, pos2[:, 0], axis=0) * top_w[:, 0:1]
           + jnp.take(ys, pos2[:, 1], axis=0) * top_w[:, 1:2])
    return out


# speedup vs baseline: 1.2793x; 1.0396x over previous
import functools

import jax
import jax.numpy as jnp
from jax.experimental import pallas as pl
from jax.experimental.pallas import tpu as pltpu

T = 2048
D = 1024
F = 4096
E = 8
K = 2

BLK = 256
P = K * T + E * BLK
NB = P // BLK
FB = 2048
NJ = F // FB

_SQRT_HALF = 0.7071067811865476


def _gelu(h):
    c = 0.7978845608028654
    return 0.5 * h * (1.0 + jnp.tanh(c * (h + 0.044715 * h * h * h)))


def _ffn_body(be_ref, xs_ref, w1_ref, b1_ref, w2_ref, b2_ref, out_ref):
    j = pl.program_id(0)
    w1 = w1_ref[0].astype(jnp.bfloat16)
    w2 = w2_ref[0].astype(jnp.bfloat16)
    h = jnp.dot(xs_ref[...], w1, preferred_element_type=jnp.float32)
    h = _gelu(h + b1_ref[0, 0]).astype(jnp.bfloat16)
    y = jnp.dot(h, w2, preferred_element_type=jnp.float32)
    out_ref[0] = jnp.where(j == 0, y + b2_ref[0, 0], y)


@jax.jit
def _ffn(xs, block_e, W1, b1, W2, b2):
    grid_spec = pltpu.PrefetchScalarGridSpec(
        num_scalar_prefetch=1,
        grid=(NJ, NB),
        in_specs=[
            pl.BlockSpec((BLK, D), lambda j, i, be: (i, 0)),
            pl.BlockSpec((1, D, FB), lambda j, i, be: (be[i], 0, j)),
            pl.BlockSpec((1, 1, FB), lambda j, i, be: (be[i], 0, j)),
            pl.BlockSpec((1, FB, D), lambda j, i, be: (be[i], j, 0)),
            pl.BlockSpec((1, 1, D), lambda j, i, be: (be[i], 0, 0)),
        ],
        out_specs=pl.BlockSpec((1, BLK, D), lambda j, i, be: (j, i, 0)),
    )
    return pl.pallas_call(
        _ffn_body,
        grid_spec=grid_spec,
        out_shape=jax.ShapeDtypeStruct((NJ, P, D), jnp.float32),
        compiler_params=pltpu.CompilerParams(
            dimension_semantics=("arbitrary", "arbitrary"),
        ),
    )(block_e, xs, W1, b1.reshape(E, 1, F), W2, b2.reshape(E, 1, D))


def kernel(x, gate_W, W1, b1, W2, b2):
    logits = x @ gate_W
    eidx = jnp.arange(E, dtype=jnp.int32)
    i1 = jnp.argmax(logits, axis=-1).astype(jnp.int32)
    m1 = jnp.max(logits, axis=-1)
    masked = jnp.where(eidx[None, :] == i1[:, None], -jnp.inf, logits)
    i2 = jnp.argmax(masked, axis=-1).astype(jnp.int32)
    m2 = jnp.max(masked, axis=-1)
    e2 = jnp.exp(m2 - m1)
    top_w = jnp.stack([1.0 / (1.0 + e2), e2 / (1.0 + e2)], axis=1)

    ef = jnp.stack([i1, i2], axis=1).reshape(-1)
    onehot = (ef[:, None] == eidx[None, :]).astype(jnp.int32)
    csum = jnp.cumsum(onehot, axis=0)
    counts = csum[-1]
    rank = jnp.take_along_axis(csum, ef[:, None], axis=1)[:, 0] - 1
    blocks_e = (counts + BLK - 1) // BLK
    first_block = jnp.concatenate(
        [jnp.zeros((1,), jnp.int32), jnp.cumsum(blocks_e)[:-1]])
    pad_start = first_block * BLK

    pos = pad_start[ef] + rank
    rowtok = jnp.zeros((P,), jnp.int32).at[pos].set(
        jnp.arange(K * T, dtype=jnp.int32) // K)

    block_e = (jnp.sum(jnp.arange(NB, dtype=jnp.int32)[:, None]
                       >= first_block[None, :], axis=1) - 1).astype(jnp.int32)

    xs = jnp.take(x.astype(jnp.bfloat16), rowtok, axis=0)
    yp = _ffn(xs, block_e, W1, b1, W2, b2)
    ys = jnp.sum(yp, axis=0)

    pos2 = pos.reshape(T, K)
    out = (jnp.take(ys, pos2[:, 0], axis=0) * top_w[:, 0:1]
           + jnp.take(ys, pos2[:, 1], axis=0) * top_w[:, 1:2])
    return out


# speedup vs baseline: 1.3409x; 1.0481x over previous
import functools

import jax
import jax.numpy as jnp
from jax.experimental import pallas as pl
from jax.experimental.pallas import tpu as pltpu

T = 2048
D = 1024
F = 4096
E = 8
K = 2

BLK = 256
P = K * T + E * BLK
NB = P // BLK
FB = 2048
NJ = F // FB

_SQRT_HALF = 0.7071067811865476


def _gelu(h):
    c = 0.7978845608028654
    return 0.5 * h * (1.0 + jnp.tanh(c * (h + 0.044715 * h * h * h)))


def _ffn_body(be_ref, xs_ref, w1_ref, b1_ref, w2_ref, b2_ref, out_ref):
    j = pl.program_id(0)
    w1 = w1_ref[0].astype(jnp.bfloat16)
    w2 = w2_ref[0].astype(jnp.bfloat16)
    h = jnp.dot(xs_ref[...], w1, preferred_element_type=jnp.float32)
    h = _gelu(h + b1_ref[0, 0]).astype(jnp.bfloat16)
    y = jnp.dot(h, w2, preferred_element_type=jnp.float32)
    out_ref[0] = jnp.where(j == 0, y + b2_ref[0, 0], y).astype(jnp.bfloat16)


@jax.jit
def _ffn(xs, block_e, W1, b1, W2, b2):
    grid_spec = pltpu.PrefetchScalarGridSpec(
        num_scalar_prefetch=1,
        grid=(NJ, NB),
        in_specs=[
            pl.BlockSpec((BLK, D), lambda j, i, be: (i, 0)),
            pl.BlockSpec((1, D, FB), lambda j, i, be: (be[i], 0, j)),
            pl.BlockSpec((1, 1, FB), lambda j, i, be: (be[i], 0, j)),
            pl.BlockSpec((1, FB, D), lambda j, i, be: (be[i], j, 0)),
            pl.BlockSpec((1, 1, D), lambda j, i, be: (be[i], 0, 0)),
        ],
        out_specs=pl.BlockSpec((1, BLK, D), lambda j, i, be: (j, i, 0)),
    )
    return pl.pallas_call(
        _ffn_body,
        grid_spec=grid_spec,
        out_shape=jax.ShapeDtypeStruct((NJ, P, D), jnp.bfloat16),
        compiler_params=pltpu.CompilerParams(
            dimension_semantics=("arbitrary", "arbitrary"),
        ),
    )(block_e, xs, W1, b1.reshape(E, 1, F), W2, b2.reshape(E, 1, D))


def kernel(x, gate_W, W1, b1, W2, b2):
    logits = x @ gate_W
    eidx = jnp.arange(E, dtype=jnp.int32)
    i1 = jnp.argmax(logits, axis=-1).astype(jnp.int32)
    m1 = jnp.max(logits, axis=-1)
    masked = jnp.where(eidx[None, :] == i1[:, None], -jnp.inf, logits)
    i2 = jnp.argmax(masked, axis=-1).astype(jnp.int32)
    m2 = jnp.max(masked, axis=-1)
    e2 = jnp.exp(m2 - m1)
    top_w = jnp.stack([1.0 / (1.0 + e2), e2 / (1.0 + e2)], axis=1)

    ef = jnp.stack([i1, i2], axis=1).reshape(-1)
    onehot = (ef[:, None] == eidx[None, :]).astype(jnp.int32)
    csum = jnp.cumsum(onehot, axis=0)
    counts = csum[-1]
    rank = jnp.take_along_axis(csum, ef[:, None], axis=1)[:, 0] - 1
    blocks_e = (counts + BLK - 1) // BLK
    first_block = jnp.concatenate(
        [jnp.zeros((1,), jnp.int32), jnp.cumsum(blocks_e)[:-1]])
    pad_start = first_block * BLK

    pos = pad_start[ef] + rank
    rowtok = jnp.zeros((P,), jnp.int32).at[pos].set(
        jnp.arange(K * T, dtype=jnp.int32) // K)

    block_e = (jnp.sum(jnp.arange(NB, dtype=jnp.int32)[:, None]
                       >= first_block[None, :], axis=1) - 1).astype(jnp.int32)

    xs = jnp.take(x.astype(jnp.bfloat16), rowtok, axis=0)
    yp = _ffn(xs, block_e, W1, b1, W2, b2)
    ys = (yp[0].astype(jnp.float32) + yp[1].astype(jnp.float32)
          ).astype(jnp.bfloat16)

    pos2 = pos.reshape(T, K)
    yt = jnp.take(ys, jnp.concatenate([pos2[:, 0], pos2[:, 1]]), axis=0)
    yt = yt.astype(jnp.float32)
    return yt[:T] * top_w[:, 0:1] + yt[T:] * top_w[:, 1:2]
